# baseline (speedup 1.0000x reference)
import jax
import jax.numpy as jnp
from jax.experimental import pallas as pl
from jax.experimental.pallas import tpu as pltpu

NUM_DRUGS = 1000
EMB_DIM = 128
BATCH = 1024
BB = 16
NG = 8
KFULL = 128 * EMB_DIM


def _decode_kernel(ids_ref, emb_ref, bias_ref, cell_ref, out_ref, w_ref):
    @pl.when(pl.program_id(0) == 0)
    def _build_w():
        ids = ids_ref[...]
        onehot = (ids == jax.lax.broadcasted_iota(
            jnp.int32, (NUM_DRUGS, NUM_DRUGS), 1)).astype(jnp.float32)
        d_tab = jax.lax.dot(
            onehot, emb_ref[...], preferred_element_type=jnp.float32)
        d_pad = jnp.concatenate(
            [d_tab, jnp.zeros((NG * 128 - NUM_DRUGS, EMB_DIM), jnp.float32)],
            axis=0)
        eye = (jax.lax.broadcasted_iota(jnp.int32, (128, 128), 0) ==
               jax.lax.broadcasted_iota(jnp.int32, (128, 128), 1)
               ).astype(jnp.float32)
        for g in range(NG):
            a_g = d_pad[g * 128:(g + 1) * 128, :].T
            w_g = a_g[None, :, :] * eye[:, None, :]
            w_ref[g, :, :] = w_g.reshape(KFULL, 128).astype(jnp.bfloat16)

    cb = cell_ref[...].astype(jnp.bfloat16)
    for g in range(NG):
        k0 = g * KFULL
        kw = KFULL if g < NG - 1 else NUM_DRUGS * EMB_DIM - k0
        mm = jax.lax.dot(
            cb[:, k0:k0 + kw], w_ref[g, :kw, :],
            preferred_element_type=jnp.float32)
        out_ref[:, g * 128:(g + 1) * 128] = mm + bias_ref[:, g * 128:(g + 1) * 128]


def kernel(cell_repr, drg_ids, emb_table, drg_bias):
    ids_col = drg_ids.astype(jnp.int32).reshape(NUM_DRUGS, 1)
    bias_pad = jnp.pad(drg_bias, (0, NG * 128 - NUM_DRUGS)).reshape(1, NG * 128)
    cell_flat = cell_repr.reshape(BATCH, NUM_DRUGS * EMB_DIM)
    grid = (BATCH // BB,)
    out = pl.pallas_call(
        _decode_kernel,
        grid=grid,
        in_specs=[
            pl.BlockSpec((NUM_DRUGS, 1), lambda i: (0, 0)),
            pl.BlockSpec((NUM_DRUGS, EMB_DIM), lambda i: (0, 0)),
            pl.BlockSpec((1, NG * 128), lambda i: (0, 0)),
            pl.BlockSpec((BB, NUM_DRUGS * EMB_DIM), lambda i: (i, 0)),
        ],
        out_specs=pl.BlockSpec((BB, NG * 128), lambda i: (i, 0)),
        out_shape=jax.ShapeDtypeStruct((BATCH, NG * 128), jnp.float32),
        scratch_shapes=[pltpu.VMEM((NG, KFULL, 128), jnp.bfloat16)],
    )(ids_col, emb_table, bias_pad, cell_flat)
    return out[:, :NUM_DRUGS]


# speedup vs baseline: 3.6663x; 3.6663x over previous
import jax
import jax.numpy as jnp
from jax.experimental import pallas as pl
from jax.experimental.pallas import tpu as pltpu

NUM_DRUGS = 1000
EMB_DIM = 128
BATCH = 1024
BB = 32
DC = 40


def _decode_kernel(ids_ref, emb_ref, bias_ref, cell_ref, out_ref, d_scratch):
    @pl.when(pl.program_id(0) == 0)
    def _gather():
        ids = ids_ref[0, :]
        onehot = (ids[:, None] == jax.lax.broadcasted_iota(
            jnp.int32, (NUM_DRUGS, NUM_DRUGS), 1)).astype(jnp.float32)
        d_scratch[...] = jax.lax.dot(
            onehot, emb_ref[...], preferred_element_type=jnp.float32)

    for j in range(NUM_DRUGS // DC):
        dd = d_scratch[j * DC:(j + 1) * DC, :]
        cols = []
        for b in range(BB):
            p = dd * cell_ref[b, j * DC:(j + 1) * DC, :]
            cols.append(jnp.sum(p, axis=1, keepdims=True))
        red_t = jnp.concatenate(cols, axis=1)
        out_ref[0, j * DC:(j + 1) * DC, :] = (
            red_t + bias_ref[j * DC:(j + 1) * DC, :])


def kernel(cell_repr, drg_ids, emb_table, drg_bias):
    ids2d = drg_ids.astype(jnp.int32).reshape(1, NUM_DRUGS)
    bias2d = drg_bias.reshape(NUM_DRUGS, 1)
    grid = (BATCH // BB,)
    out_t = pl.pallas_call(
        _decode_kernel,
        grid=grid,
        in_specs=[
            pl.BlockSpec((1, NUM_DRUGS), lambda i: (0, 0)),
            pl.BlockSpec((NUM_DRUGS, EMB_DIM), lambda i: (0, 0)),
            pl.BlockSpec((NUM_DRUGS, 1), lambda i: (0, 0)),
            pl.BlockSpec((BB, NUM_DRUGS, EMB_DIM), lambda i: (i, 0, 0)),
        ],
        out_specs=pl.BlockSpec((1, NUM_DRUGS, BB), lambda i: (i, 0, 0)),
        out_shape=jax.ShapeDtypeStruct((BATCH // BB, NUM_DRUGS, BB), jnp.float32),
        scratch_shapes=[pltpu.VMEM((NUM_DRUGS, EMB_DIM), jnp.float32)],
    )(ids2d, emb_table, bias2d, cell_repr)
    return out_t.transpose(0, 2, 1).reshape(BATCH, NUM_DRUGS)
